# baseline (speedup 1.0000x reference)
import jax
import jax.numpy as jnp
from jax.experimental import pallas as pl

N_S = 10000
N_H = 10000
DIM = 128
RANK = 64
LAYER = 2
TEMP = 0.2
LAMBDA_1 = 0.2
LAMBDA_2 = 1e-07
BN_EPS = 1e-05
B = 4096
BU = 1024

K_PAD = 10240
K_TILE = 1024


def _spmm(rows, cols, vals, X, n_out):
    return jax.ops.segment_sum(vals[:, None] * X[cols], rows, num_segments=n_out)


def _matmul_kernel(a_ref, b_ref, o_ref):
    @pl.when(pl.program_id(0) == 0)
    def _():
        o_ref[...] = jnp.zeros_like(o_ref)

    o_ref[...] += jnp.dot(a_ref[...], b_ref[...],
                          preferred_element_type=jnp.float32)


def _pallas_matmul(a, b):
    M, K = a.shape
    _, N = b.shape
    grid = (K // K_TILE,)
    return pl.pallas_call(
        _matmul_kernel,
        grid=grid,
        in_specs=[
            pl.BlockSpec((M, K_TILE), lambda k: (0, k)),
            pl.BlockSpec((K_TILE, N), lambda k: (k, 0)),
        ],
        out_specs=pl.BlockSpec((M, N), lambda k: (0, 0)),
        out_shape=jax.ShapeDtypeStruct((M, N), jnp.float32),
    )(a, b)


def kernel(sids, hids, pos, neg, ps, E_s_0, E_h_0, E_ss_0, E_hh_0,
           adj_rows, adj_cols, adj_vals, ss_rows, ss_cols, ss_vals,
           hh_rows, hh_cols, hh_vals, u_mul_s, vt, v_mul_s, ut,
           bn_gamma, bn_beta):
    E_s_list = [E_s_0]
    E_h_list = [E_h_0]
    E_ss_list = [E_ss_0]
    E_hh_list = [E_hh_0]
    G_s_list = [E_s_0]
    G_h_list = [E_h_0]
    for _ in range(LAYER):
        Z_s = _spmm(adj_rows, adj_cols, adj_vals, E_h_list[-1], N_S)
        Z_h = _spmm(adj_cols, adj_rows, adj_vals, E_s_list[-1], N_H)
        Z_ss = _spmm(ss_rows, ss_cols, ss_vals, E_ss_list[-1], N_S)
        Z_hh = _spmm(hh_rows, hh_cols, hh_vals, E_hh_list[-1], N_H)
        vt_eh = vt @ E_h_list[-1]
        G_s_list.append(u_mul_s @ vt_eh)
        ut_es = ut @ E_s_list[-1]
        G_h_list.append(v_mul_s @ ut_es)
        E_s_list.append(Z_s)
        E_h_list.append(Z_h)
        E_ss_list.append(Z_ss)
        E_hh_list.append(Z_hh)
    G_s = sum(G_s_list)
    G_h = sum(G_h_list)
    E_s = sum(E_s_list)
    E_h = sum(E_h_list)
    E_ss = sum(E_ss_list)
    E_hh = sum(E_hh_list)

    ps_pad = jnp.pad(ps, ((0, 0), (0, K_PAD - N_S)))
    Es_sum_pad = jnp.pad(E_s + E_ss, ((0, K_PAD - N_S), (0, 0)))
    e_synd = _pallas_matmul(ps_pad, Es_sum_pad)

    preSum = jnp.sum(ps, axis=1, keepdims=True)
    e = e_synd / preSum
    mean = jnp.mean(e, axis=0)
    var = jnp.var(e, axis=0)
    e = (e - mean) / jnp.sqrt(var + BN_EPS) * bn_gamma + bn_beta
    e = jax.nn.relu(e)
    pre = e @ (E_h + E_hh).T
    neg_score = jnp.log(jnp.sum(jnp.exp(G_s[sids] @ E_s.T / TEMP), axis=1) + 1e-08).mean()
    neg_score = neg_score + jnp.log(jnp.sum(jnp.exp(G_h[hids] @ E_h.T / TEMP), axis=1) + 1e-08).mean()
    pos_score = jnp.clip(jnp.sum(G_s[sids] * E_s[sids], axis=1) / TEMP, -5.0, 5.0).mean() \
        + jnp.clip(jnp.sum(G_h[hids] * E_h[hids], axis=1) / TEMP, -5.0, 5.0).mean()
    loss_s = -pos_score + neg_score
    s_emb = E_s[sids]
    pos_emb = E_h[pos]
    neg_emb = E_h[neg]
    pos_scores = jnp.sum(s_emb * pos_emb, axis=-1)
    neg_scores = jnp.sum(s_emb * neg_emb, axis=-1)
    loss_r = -jnp.log(jax.nn.sigmoid(pos_scores - neg_scores)).mean()
    loss_reg = jnp.float32(0.0)
    for p in [E_s_0, E_h_0, E_ss_0, E_hh_0, bn_gamma, bn_beta]:
        loss_reg = loss_reg + jnp.square(jnp.linalg.norm(p))
    loss_reg = loss_reg * LAMBDA_2
    loss = loss_r + LAMBDA_1 * loss_s + loss_reg
    return (loss, loss_r, LAMBDA_1 * loss_s, pre)


# speedup vs baseline: 1.8437x; 1.8437x over previous
import functools

import jax
import jax.numpy as jnp
from jax import lax
from jax.experimental import pallas as pl
from jax.experimental.pallas import tpu as pltpu
from jax.experimental.pallas import tpu_sc as plsc

N_S = 10000
N_H = 10000
DIM = 128
RANK = 64
LAYER = 2
TEMP = 0.2
LAMBDA_1 = 0.2
LAMBDA_2 = 1e-07
BN_EPS = 1e-05
B = 4096
BU = 1024

E_EDGES = 320000
CHUNK = 128
N_CHUNKS = 2560
CHUNKS_PER_TILE = N_CHUNKS // 16
ROWS_MAIN = 624
ROWS_LAST = N_S - 15 * ROWS_MAIN

K_PAD = 10240
K_TILE = 1024

_MESH = plsc.VectorSubcoreMesh(core_axis_name="c", subcore_axis_name="s")


def _zero_zbuf(zbuf):
    zeros16 = jnp.zeros((16,), jnp.float32)

    def body(r, _):
        for d in range(8):
            zbuf[r, pl.ds(d * 16, 16)] = zeros16
        return 0

    lax.fori_loop(0, 128, body, 0)


def _do_spmm(rows2, cols2, vals2, x_hbm, z_hbm, acc, cols_v, ridx_v, vals_v,
             rbuf, zbuf, sem, sid):
    base_row = sid * ROWS_MAIN

    @pl.when(sid < 15)
    def _():
        for k in range(4):
            pltpu.sync_copy(zbuf, acc.at[pl.ds(base_row + k * 128, 128)])
        pltpu.sync_copy(zbuf.at[pl.ds(0, ROWS_MAIN - 512)],
                        acc.at[pl.ds(base_row + 512, ROWS_MAIN - 512)])

    @pl.when(sid == 15)
    def _():
        for k in range(5):
            pltpu.sync_copy(zbuf, acc.at[pl.ds(15 * ROWS_MAIN + k * 128, 128)])

    plsc.subcore_barrier()

    base_chunk = sid * CHUNKS_PER_TILE

    def chunk_body(i, _):
        j = base_chunk + i
        pltpu.sync_copy(cols2.at[j], cols_v)
        pltpu.sync_copy(vals2.at[j], vals_v)
        pltpu.async_copy(x_hbm.at[cols_v], rbuf, sem).wait()

        def scale_body(g, _):
            vv = vals_v[pl.ds(g * 16, 16)]
            for l in range(16):
                v = vv[l]
                e = g * 16 + l
                for d in range(8):
                    sl = pl.ds(d * 16, 16)
                    rbuf[e, sl] = rbuf[e, sl] * v
            return 0

        lax.fori_loop(0, CHUNK // 16, scale_body, 0)
        pltpu.sync_copy(rows2.at[j], ridx_v)
        pltpu.sync_copy(rbuf, acc.at[ridx_v], add=True)
        return 0

    lax.fori_loop(0, CHUNKS_PER_TILE, chunk_body, 0)
    plsc.subcore_barrier()

    @pl.when(sid < 15)
    def _():
        for k in range(4):
            pltpu.sync_copy(acc.at[pl.ds(base_row + k * 128, 128)],
                            z_hbm.at[pl.ds(base_row + k * 128, 128)])
        pltpu.sync_copy(acc.at[pl.ds(base_row + 512, ROWS_MAIN - 512)],
                        z_hbm.at[pl.ds(base_row + 512, ROWS_MAIN - 512)])

    @pl.when(sid == 15)
    def _():
        for k in range(5):
            pltpu.sync_copy(acc.at[pl.ds(15 * ROWS_MAIN + k * 128, 128)],
                            z_hbm.at[pl.ds(15 * ROWS_MAIN + k * 128, 128)])

    plsc.subcore_barrier()


@functools.partial(
    pl.kernel,
    mesh=_MESH,
    out_type=[jax.ShapeDtypeStruct((N_S, DIM), jnp.float32),
              jax.ShapeDtypeStruct((N_H, DIM), jnp.float32),
              jax.ShapeDtypeStruct((N_S, DIM), jnp.float32),
              jax.ShapeDtypeStruct((N_H, DIM), jnp.float32)],
    scratch_types=[
        pltpu.VMEM_SHARED((N_S, DIM), jnp.float32),
        pltpu.VMEM((CHUNK,), jnp.int32),
        pltpu.VMEM((CHUNK,), jnp.int32),
        pltpu.VMEM((CHUNK,), jnp.float32),
        pltpu.VMEM((CHUNK, DIM), jnp.float32),
        pltpu.VMEM((CHUNK, DIM), jnp.float32),
        pltpu.SemaphoreType.DMA,
    ],
)
def _sc_layer(adj_r2, adj_c2, adj_v2, ss_r2, ss_c2, ss_v2,
              hh_r2, hh_c2, hh_v2, xs, xh, xss, xhh,
              z_s, z_h, z_ss, z_hh,
              acc, cols_v, ridx_v, vals_v, rbuf, zbuf, sem):
    cid = lax.axis_index("c")
    sid = lax.axis_index("s")
    _zero_zbuf(zbuf)

    @pl.when(cid == 0)
    def _():
        _do_spmm(adj_r2, adj_c2, adj_v2, xh, z_s, acc, cols_v, ridx_v,
                 vals_v, rbuf, zbuf, sem, sid)
        _do_spmm(ss_r2, ss_c2, ss_v2, xss, z_ss, acc, cols_v, ridx_v,
                 vals_v, rbuf, zbuf, sem, sid)

    @pl.when(cid == 1)
    def _():
        _do_spmm(adj_c2, adj_r2, adj_v2, xs, z_h, acc, cols_v, ridx_v,
                 vals_v, rbuf, zbuf, sem, sid)
        _do_spmm(hh_r2, hh_c2, hh_v2, xhh, z_hh, acc, cols_v, ridx_v,
                 vals_v, rbuf, zbuf, sem, sid)


def _pad_edges(rows, cols, vals):
    pad = N_CHUNKS * CHUNK - E_EDGES
    r = jnp.pad(rows.astype(jnp.int32), (0, pad)).reshape(N_CHUNKS, CHUNK)
    c = jnp.pad(cols.astype(jnp.int32), (0, pad)).reshape(N_CHUNKS, CHUNK)
    v = jnp.pad(vals, (0, pad)).reshape(N_CHUNKS, CHUNK)
    return r, c, v


def _matmul_kernel(a_ref, b_ref, o_ref):
    @pl.when(pl.program_id(0) == 0)
    def _():
        o_ref[...] = jnp.zeros_like(o_ref)

    o_ref[...] += jnp.dot(a_ref[...], b_ref[...],
                          preferred_element_type=jnp.float32)


def _pallas_matmul(a, b):
    M, K = a.shape
    _, N = b.shape
    grid = (K // K_TILE,)
    return pl.pallas_call(
        _matmul_kernel,
        grid=grid,
        in_specs=[
            pl.BlockSpec((M, K_TILE), lambda k: (0, k)),
            pl.BlockSpec((K_TILE, N), lambda k: (k, 0)),
        ],
        out_specs=pl.BlockSpec((M, N), lambda k: (0, 0)),
        out_shape=jax.ShapeDtypeStruct((M, N), jnp.float32),
    )(a, b)


def kernel(sids, hids, pos, neg, ps, E_s_0, E_h_0, E_ss_0, E_hh_0,
           adj_rows, adj_cols, adj_vals, ss_rows, ss_cols, ss_vals,
           hh_rows, hh_cols, hh_vals, u_mul_s, vt, v_mul_s, ut,
           bn_gamma, bn_beta):
    adj_r2, adj_c2, adj_v2 = _pad_edges(adj_rows, adj_cols, adj_vals)
    ss_r2, ss_c2, ss_v2 = _pad_edges(ss_rows, ss_cols, ss_vals)
    hh_r2, hh_c2, hh_v2 = _pad_edges(hh_rows, hh_cols, hh_vals)

    Z_s1, Z_h1, Z_ss1, Z_hh1 = _sc_layer(
        adj_r2, adj_c2, adj_v2, ss_r2, ss_c2, ss_v2, hh_r2, hh_c2, hh_v2,
        E_s_0, E_h_0, E_ss_0, E_hh_0)
    Z_s2, Z_h2, Z_ss2, Z_hh2 = _sc_layer(
        adj_r2, adj_c2, adj_v2, ss_r2, ss_c2, ss_v2, hh_r2, hh_c2, hh_v2,
        Z_s1, Z_h1, Z_ss1, Z_hh1)

    G_s = E_s_0 + u_mul_s @ (vt @ (E_h_0 + Z_h1))
    G_h = E_h_0 + v_mul_s @ (ut @ (E_s_0 + Z_s1))
    E_s = E_s_0 + Z_s1 + Z_s2
    E_h = E_h_0 + Z_h1 + Z_h2
    E_ss = E_ss_0 + Z_ss1 + Z_ss2
    E_hh = E_hh_0 + Z_hh1 + Z_hh2

    ps_pad = jnp.pad(ps, ((0, 0), (0, K_PAD - N_S)))
    Es_sum_pad = jnp.pad(E_s + E_ss, ((0, K_PAD - N_S), (0, 0)))
    e_synd = _pallas_matmul(ps_pad, Es_sum_pad)

    preSum = jnp.sum(ps, axis=1, keepdims=True)
    e = e_synd / preSum
    mean = jnp.mean(e, axis=0)
    var = jnp.var(e, axis=0)
    e = (e - mean) / jnp.sqrt(var + BN_EPS) * bn_gamma + bn_beta
    e = jax.nn.relu(e)
    pre = e @ (E_h + E_hh).T
    neg_score = jnp.log(jnp.sum(jnp.exp(G_s[sids] @ E_s.T / TEMP), axis=1) + 1e-08).mean()
    neg_score = neg_score + jnp.log(jnp.sum(jnp.exp(G_h[hids] @ E_h.T / TEMP), axis=1) + 1e-08).mean()
    pos_score = jnp.clip(jnp.sum(G_s[sids] * E_s[sids], axis=1) / TEMP, -5.0, 5.0).mean() \
        + jnp.clip(jnp.sum(G_h[hids] * E_h[hids], axis=1) / TEMP, -5.0, 5.0).mean()
    loss_s = -pos_score + neg_score
    s_emb = E_s[sids]
    pos_emb = E_h[pos]
    neg_emb = E_h[neg]
    pos_scores = jnp.sum(s_emb * pos_emb, axis=-1)
    neg_scores = jnp.sum(s_emb * neg_emb, axis=-1)
    loss_r = -jnp.log(jax.nn.sigmoid(pos_scores - neg_scores)).mean()
    loss_reg = jnp.float32(0.0)
    for p in [E_s_0, E_h_0, E_ss_0, E_hh_0, bn_gamma, bn_beta]:
        loss_reg = loss_reg + jnp.square(jnp.linalg.norm(p))
    loss_reg = loss_reg * LAMBDA_2
    loss = loss_r + LAMBDA_1 * loss_s + loss_reg
    return (loss, loss_r, LAMBDA_1 * loss_s, pre)


# speedup vs baseline: 2.9332x; 1.5909x over previous
import functools

import jax
import jax.numpy as jnp
from jax import lax
from jax.experimental import pallas as pl
from jax.experimental.pallas import tpu as pltpu
from jax.experimental.pallas import tpu_sc as plsc

N_S = 10000
N_H = 10000
DIM = 128
RANK = 64
LAYER = 2
TEMP = 0.2
LAMBDA_1 = 0.2
LAMBDA_2 = 1e-07
BN_EPS = 1e-05
B = 4096
BU = 1024

E_EDGES = 320000
CHUNK = 128
N_CHUNKS = 2560
CHUNKS_PER_TILE = N_CHUNKS // 16
CHUNKS_PER_BLOCK = 32
N_BLOCKS = CHUNKS_PER_TILE // CHUNKS_PER_BLOCK
ROWS_MAIN = 624
ROWS_LAST = N_S - 15 * ROWS_MAIN

K_PAD = 10240
K_TILE = 1024

_MESH = plsc.VectorSubcoreMesh(core_axis_name="c", subcore_axis_name="s")


def _fill_zeros(rbuf):
    zeros16 = jnp.zeros((16,), jnp.float32)

    def body(r, _):
        for d in range(8):
            rbuf[r, pl.ds(d * 16, 16)] = zeros16
        return 0

    lax.fori_loop(0, CHUNK, body, 0)


def _do_spmm(rows2, cols2, vals2, x_hbm, z_hbm, acc, colsb, rowsb, valsb,
             rbuf0, rbuf1, sem_g0, sem_g1, sem_s0, sem_s1, sid):
    base_row = sid * ROWS_MAIN

    _fill_zeros(rbuf0)

    @pl.when(sid < 15)
    def _():
        for k in range(4):
            pltpu.sync_copy(rbuf0, acc.at[pl.ds(base_row + k * 128, 128)])
        pltpu.sync_copy(rbuf0.at[pl.ds(0, ROWS_MAIN - 512)],
                        acc.at[pl.ds(base_row + 512, ROWS_MAIN - 512)])

    @pl.when(sid == 15)
    def _():
        for k in range(5):
            pltpu.sync_copy(rbuf0, acc.at[pl.ds(15 * ROWS_MAIN + k * 128, 128)])

    plsc.subcore_barrier()

    bufs = (rbuf0, rbuf1)
    gsems = (sem_g0, sem_g1)
    ssems = (sem_s0, sem_s1)

    def gather_start(j, p):
        pltpu.async_copy(x_hbm.at[colsb.at[j]], bufs[p], gsems[p])

    def gather_wait(j, p):
        pltpu.make_async_copy(x_hbm.at[colsb.at[j]], bufs[p], gsems[p]).wait()

    def scatter_start(j, p):
        pltpu.async_copy(bufs[p], acc.at[rowsb.at[j]], ssems[p], add=True)

    def scatter_wait(j, p):
        pltpu.make_async_copy(bufs[p], acc.at[rowsb.at[j]], ssems[p]).wait()

    def scale(j, p):
        rbuf = bufs[p]

        def scale_body(g, _):
            vv = valsb[j, pl.ds(g * 16, 16)]
            for l in range(16):
                v = vv[l]
                e = g * 16 + l
                for d in range(8):
                    sl = pl.ds(d * 16, 16)
                    rbuf[e, sl] = rbuf[e, sl] * v
            return 0

        lax.fori_loop(0, CHUNK // 16, scale_body, 0)

    def half(j, p):
        q = 1 - p
        gather_wait(j, p)

        @pl.when(j > 0)
        def _():
            scatter_wait(j - 1, q)

        @pl.when(j + 1 < CHUNKS_PER_BLOCK)
        def _():
            gather_start(j + 1, q)

        scale(j, p)
        scatter_start(j, p)

    base_chunk = sid * CHUNKS_PER_TILE

    def block_body(blk, _):
        off = base_chunk + blk * CHUNKS_PER_BLOCK
        pltpu.sync_copy(cols2.at[pl.ds(off, CHUNKS_PER_BLOCK)], colsb)
        pltpu.sync_copy(rows2.at[pl.ds(off, CHUNKS_PER_BLOCK)], rowsb)
        pltpu.sync_copy(vals2.at[pl.ds(off, CHUNKS_PER_BLOCK)], valsb)

        gather_start(0, 0)

        def chunk_body(i2, _):
            half(2 * i2, 0)
            half(2 * i2 + 1, 1)
            return 0

        lax.fori_loop(0, CHUNKS_PER_BLOCK // 2, chunk_body, 0)
        scatter_wait(CHUNKS_PER_BLOCK - 1, 1)
        return 0

    lax.fori_loop(0, N_BLOCKS, block_body, 0)
    plsc.subcore_barrier()

    @pl.when(sid < 15)
    def _():
        for k in range(4):
            pltpu.sync_copy(acc.at[pl.ds(base_row + k * 128, 128)],
                            z_hbm.at[pl.ds(base_row + k * 128, 128)])
        pltpu.sync_copy(acc.at[pl.ds(base_row + 512, ROWS_MAIN - 512)],
                        z_hbm.at[pl.ds(base_row + 512, ROWS_MAIN - 512)])

    @pl.when(sid == 15)
    def _():
        for k in range(5):
            pltpu.sync_copy(acc.at[pl.ds(15 * ROWS_MAIN + k * 128, 128)],
                            z_hbm.at[pl.ds(15 * ROWS_MAIN + k * 128, 128)])

    plsc.subcore_barrier()


@functools.partial(
    pl.kernel,
    mesh=_MESH,
    out_type=[jax.ShapeDtypeStruct((N_S, DIM), jnp.float32),
              jax.ShapeDtypeStruct((N_H, DIM), jnp.float32),
              jax.ShapeDtypeStruct((N_S, DIM), jnp.float32),
              jax.ShapeDtypeStruct((N_H, DIM), jnp.float32)],
    scratch_types=[
        pltpu.VMEM_SHARED((N_S, DIM), jnp.float32),
        pltpu.VMEM((CHUNKS_PER_BLOCK, CHUNK), jnp.int32),
        pltpu.VMEM((CHUNKS_PER_BLOCK, CHUNK), jnp.int32),
        pltpu.VMEM((CHUNKS_PER_BLOCK, CHUNK), jnp.float32),
        pltpu.VMEM((CHUNK, DIM), jnp.float32),
        pltpu.VMEM((CHUNK, DIM), jnp.float32),
        pltpu.SemaphoreType.DMA,
        pltpu.SemaphoreType.DMA,
        pltpu.SemaphoreType.DMA,
        pltpu.SemaphoreType.DMA,
    ],
)
def _sc_layer(adj_r2, adj_c2, adj_v2, ss_r2, ss_c2, ss_v2,
              hh_r2, hh_c2, hh_v2, xs, xh, xss, xhh,
              z_s, z_h, z_ss, z_hh,
              acc, colsb, rowsb, valsb, rbuf0, rbuf1,
              sem_g0, sem_g1, sem_s0, sem_s1):
    cid = lax.axis_index("c")
    sid = lax.axis_index("s")

    @pl.when(cid == 0)
    def _():
        _do_spmm(adj_r2, adj_c2, adj_v2, xh, z_s, acc, colsb, rowsb, valsb,
                 rbuf0, rbuf1, sem_g0, sem_g1, sem_s0, sem_s1, sid)
        _do_spmm(ss_r2, ss_c2, ss_v2, xss, z_ss, acc, colsb, rowsb, valsb,
                 rbuf0, rbuf1, sem_g0, sem_g1, sem_s0, sem_s1, sid)

    @pl.when(cid == 1)
    def _():
        _do_spmm(adj_c2, adj_r2, adj_v2, xs, z_h, acc, colsb, rowsb, valsb,
                 rbuf0, rbuf1, sem_g0, sem_g1, sem_s0, sem_s1, sid)
        _do_spmm(hh_r2, hh_c2, hh_v2, xhh, z_hh, acc, colsb, rowsb, valsb,
                 rbuf0, rbuf1, sem_g0, sem_g1, sem_s0, sem_s1, sid)


def _pad_edges(rows, cols, vals):
    pad = N_CHUNKS * CHUNK - E_EDGES
    r = jnp.pad(rows.astype(jnp.int32), (0, pad)).reshape(N_CHUNKS, CHUNK)
    c = jnp.pad(cols.astype(jnp.int32), (0, pad)).reshape(N_CHUNKS, CHUNK)
    v = jnp.pad(vals, (0, pad)).reshape(N_CHUNKS, CHUNK)
    return r, c, v


def _matmul_kernel(a_ref, b_ref, o_ref):
    @pl.when(pl.program_id(0) == 0)
    def _():
        o_ref[...] = jnp.zeros_like(o_ref)

    o_ref[...] += jnp.dot(a_ref[...], b_ref[...],
                          preferred_element_type=jnp.float32)


def _pallas_matmul(a, b):
    M, K = a.shape
    _, N = b.shape
    grid = (K // K_TILE,)
    return pl.pallas_call(
        _matmul_kernel,
        grid=grid,
        in_specs=[
            pl.BlockSpec((M, K_TILE), lambda k: (0, k)),
            pl.BlockSpec((K_TILE, N), lambda k: (k, 0)),
        ],
        out_specs=pl.BlockSpec((M, N), lambda k: (0, 0)),
        out_shape=jax.ShapeDtypeStruct((M, N), jnp.float32),
    )(a, b)


def kernel(sids, hids, pos, neg, ps, E_s_0, E_h_0, E_ss_0, E_hh_0,
           adj_rows, adj_cols, adj_vals, ss_rows, ss_cols, ss_vals,
           hh_rows, hh_cols, hh_vals, u_mul_s, vt, v_mul_s, ut,
           bn_gamma, bn_beta):
    adj_r2, adj_c2, adj_v2 = _pad_edges(adj_rows, adj_cols, adj_vals)
    ss_r2, ss_c2, ss_v2 = _pad_edges(ss_rows, ss_cols, ss_vals)
    hh_r2, hh_c2, hh_v2 = _pad_edges(hh_rows, hh_cols, hh_vals)

    Z_s1, Z_h1, Z_ss1, Z_hh1 = _sc_layer(
        adj_r2, adj_c2, adj_v2, ss_r2, ss_c2, ss_v2, hh_r2, hh_c2, hh_v2,
        E_s_0, E_h_0, E_ss_0, E_hh_0)
    Z_s2, Z_h2, Z_ss2, Z_hh2 = _sc_layer(
        adj_r2, adj_c2, adj_v2, ss_r2, ss_c2, ss_v2, hh_r2, hh_c2, hh_v2,
        Z_s1, Z_h1, Z_ss1, Z_hh1)

    G_s = E_s_0 + u_mul_s @ (vt @ (E_h_0 + Z_h1))
    G_h = E_h_0 + v_mul_s @ (ut @ (E_s_0 + Z_s1))
    E_s = E_s_0 + Z_s1 + Z_s2
    E_h = E_h_0 + Z_h1 + Z_h2
    E_ss = E_ss_0 + Z_ss1 + Z_ss2
    E_hh = E_hh_0 + Z_hh1 + Z_hh2

    ps_pad = jnp.pad(ps, ((0, 0), (0, K_PAD - N_S)))
    Es_sum_pad = jnp.pad(E_s + E_ss, ((0, K_PAD - N_S), (0, 0)))
    e_synd = _pallas_matmul(ps_pad, Es_sum_pad)

    preSum = jnp.sum(ps, axis=1, keepdims=True)
    e = e_synd / preSum
    mean = jnp.mean(e, axis=0)
    var = jnp.var(e, axis=0)
    e = (e - mean) / jnp.sqrt(var + BN_EPS) * bn_gamma + bn_beta
    e = jax.nn.relu(e)
    pre = e @ (E_h + E_hh).T
    neg_score = jnp.log(jnp.sum(jnp.exp(G_s[sids] @ E_s.T / TEMP), axis=1) + 1e-08).mean()
    neg_score = neg_score + jnp.log(jnp.sum(jnp.exp(G_h[hids] @ E_h.T / TEMP), axis=1) + 1e-08).mean()
    pos_score = jnp.clip(jnp.sum(G_s[sids] * E_s[sids], axis=1) / TEMP, -5.0, 5.0).mean() \
        + jnp.clip(jnp.sum(G_h[hids] * E_h[hids], axis=1) / TEMP, -5.0, 5.0).mean()
    loss_s = -pos_score + neg_score
    s_emb = E_s[sids]
    pos_emb = E_h[pos]
    neg_emb = E_h[neg]
    pos_scores = jnp.sum(s_emb * pos_emb, axis=-1)
    neg_scores = jnp.sum(s_emb * neg_emb, axis=-1)
    loss_r = -jnp.log(jax.nn.sigmoid(pos_scores - neg_scores)).mean()
    loss_reg = jnp.float32(0.0)
    for p in [E_s_0, E_h_0, E_ss_0, E_hh_0, bn_gamma, bn_beta]:
        loss_reg = loss_reg + jnp.square(jnp.linalg.norm(p))
    loss_reg = loss_reg * LAMBDA_2
    loss = loss_r + LAMBDA_1 * loss_s + loss_reg
    return (loss, loss_r, LAMBDA_1 * loss_s, pre)


# speedup vs baseline: 3.1092x; 1.0600x over previous
import functools

import jax
import jax.numpy as jnp
from jax import lax
from jax.experimental import pallas as pl
from jax.experimental.pallas import tpu as pltpu
from jax.experimental.pallas import tpu_sc as plsc

N_S = 10000
N_H = 10000
DIM = 128
RANK = 64
LAYER = 2
TEMP = 0.2
LAMBDA_1 = 0.2
LAMBDA_2 = 1e-07
BN_EPS = 1e-05
B = 4096
BU = 1024

E_EDGES = 320000
CHUNK = 128
N_CHUNKS = 2560
CHUNKS_PER_TILE = N_CHUNKS // 16
CHUNKS_PER_BLOCK = 32
N_BLOCKS = CHUNKS_PER_TILE // CHUNKS_PER_BLOCK
ROWS_MAIN = 624
ROWS_LAST = N_S - 15 * ROWS_MAIN

K_PAD = 10240
K_TILE = 1024

_MESH = plsc.VectorSubcoreMesh(core_axis_name="c", subcore_axis_name="s")


def _fill_zeros(rbuf):
    zeros16 = jnp.zeros((16,), jnp.float32)

    def body(r, _):
        for d in range(8):
            rbuf[r, pl.ds(d * 16, 16)] = zeros16
        return 0

    lax.fori_loop(0, CHUNK, body, 0)


def _do_spmm(rows2, cols2, vals2, x_hbm, z_hbm, acc, colsb, rowsb, valsb,
             rbuf0, rbuf1, sem_g0, sem_g1, sem_s0, sem_s1, sid):
    base_row = sid * ROWS_MAIN

    _fill_zeros(rbuf0)

    @pl.when(sid < 15)
    def _():
        for k in range(4):
            pltpu.sync_copy(rbuf0, acc.at[pl.ds(base_row + k * 128, 128)])
        pltpu.sync_copy(rbuf0.at[pl.ds(0, ROWS_MAIN - 512)],
                        acc.at[pl.ds(base_row + 512, ROWS_MAIN - 512)])

    @pl.when(sid == 15)
    def _():
        for k in range(5):
            pltpu.sync_copy(rbuf0, acc.at[pl.ds(15 * ROWS_MAIN + k * 128, 128)])

    plsc.subcore_barrier()

    bufs = (rbuf0, rbuf1)
    gsems = (sem_g0, sem_g1)
    ssems = (sem_s0, sem_s1)

    def gather_start(j, p):
        pltpu.async_copy(x_hbm.at[colsb.at[j]], bufs[p], gsems[p])

    def gather_wait(j, p):
        pltpu.make_async_copy(x_hbm.at[colsb.at[j]], bufs[p], gsems[p]).wait()

    def scatter_start(j, p):
        pltpu.async_copy(bufs[p], acc.at[rowsb.at[j]], ssems[p], add=True)

    def scatter_wait(j, p):
        pltpu.make_async_copy(bufs[p], acc.at[rowsb.at[j]], ssems[p]).wait()

    def scale(j, p):
        rbuf = bufs[p]

        def scale_body(g, _):
            vv = valsb[j, pl.ds(g * 16, 16)]
            for l in range(16):
                v = vv[l]
                e = g * 16 + l
                for d in range(8):
                    sl = pl.ds(d * 16, 16)
                    rbuf[e, sl] = rbuf[e, sl] * v
            return 0

        lax.fori_loop(0, CHUNK // 16, scale_body, 0)

    def half(j, p):
        q = 1 - p
        gather_wait(j, p)

        @pl.when(j > 0)
        def _():
            scatter_wait(j - 1, q)

        @pl.when(j + 1 < CHUNKS_PER_BLOCK)
        def _():
            gather_start(j + 1, q)

        scale(j, p)
        scatter_start(j, p)

    base_chunk = sid * CHUNKS_PER_TILE

    def block_body(blk, _):
        off = base_chunk + blk * CHUNKS_PER_BLOCK
        pltpu.sync_copy(cols2.at[pl.ds(off, CHUNKS_PER_BLOCK)], colsb)
        pltpu.sync_copy(rows2.at[pl.ds(off, CHUNKS_PER_BLOCK)], rowsb)
        pltpu.sync_copy(vals2.at[pl.ds(off, CHUNKS_PER_BLOCK)], valsb)

        gather_start(0, 0)

        def chunk_body(i2, _):
            half(2 * i2, 0)
            half(2 * i2 + 1, 1)
            return 0

        lax.fori_loop(0, CHUNKS_PER_BLOCK // 2, chunk_body, 0)
        scatter_wait(CHUNKS_PER_BLOCK - 1, 1)
        return 0

    lax.fori_loop(0, N_BLOCKS, block_body, 0)
    plsc.subcore_barrier()

    @pl.when(sid < 15)
    def _():
        for k in range(4):
            pltpu.sync_copy(acc.at[pl.ds(base_row + k * 128, 128)],
                            z_hbm.at[pl.ds(base_row + k * 128, 128)])
        pltpu.sync_copy(acc.at[pl.ds(base_row + 512, ROWS_MAIN - 512)],
                        z_hbm.at[pl.ds(base_row + 512, ROWS_MAIN - 512)])

    @pl.when(sid == 15)
    def _():
        for k in range(5):
            pltpu.sync_copy(acc.at[pl.ds(15 * ROWS_MAIN + k * 128, 128)],
                            z_hbm.at[pl.ds(15 * ROWS_MAIN + k * 128, 128)])

    plsc.subcore_barrier()


@functools.partial(
    pl.kernel,
    mesh=_MESH,
    out_type=[jax.ShapeDtypeStruct((N_S, DIM), jnp.float32),
              jax.ShapeDtypeStruct((N_H, DIM), jnp.float32),
              jax.ShapeDtypeStruct((N_S, DIM), jnp.float32),
              jax.ShapeDtypeStruct((N_H, DIM), jnp.float32),
              jax.ShapeDtypeStruct((N_S, DIM), jnp.float32),
              jax.ShapeDtypeStruct((N_H, DIM), jnp.float32),
              jax.ShapeDtypeStruct((N_S, DIM), jnp.float32),
              jax.ShapeDtypeStruct((N_H, DIM), jnp.float32)],
    scratch_types=[
        pltpu.VMEM_SHARED((N_S, DIM), jnp.float32),
        pltpu.VMEM((CHUNKS_PER_BLOCK, CHUNK), jnp.int32),
        pltpu.VMEM((CHUNKS_PER_BLOCK, CHUNK), jnp.int32),
        pltpu.VMEM((CHUNKS_PER_BLOCK, CHUNK), jnp.float32),
        pltpu.VMEM((CHUNK, DIM), jnp.float32),
        pltpu.VMEM((CHUNK, DIM), jnp.float32),
        pltpu.SemaphoreType.DMA,
        pltpu.SemaphoreType.DMA,
        pltpu.SemaphoreType.DMA,
        pltpu.SemaphoreType.DMA,
    ],
)
def _sc_all(adj_r2, adj_c2, adj_v2, ss_r2, ss_c2, ss_v2,
            hh_r2, hh_c2, hh_v2, es0, eh0, ess0, ehh0,
            z_s1, z_h1, z_ss1, z_hh1, z_s2, z_h2, z_ss2, z_hh2,
            acc, colsb, rowsb, valsb, rbuf0, rbuf1,
            sem_g0, sem_g1, sem_s0, sem_s1):
    cid = lax.axis_index("c")
    sid = lax.axis_index("s")

    def spmm(rows2, cols2, vals2, x_hbm, z_hbm):
        _do_spmm(rows2, cols2, vals2, x_hbm, z_hbm, acc, colsb, rowsb, valsb,
                 rbuf0, rbuf1, sem_g0, sem_g1, sem_s0, sem_s1, sid)

    @pl.when(cid == 0)
    def _():
        spmm(adj_r2, adj_c2, adj_v2, eh0, z_s1)
        spmm(ss_r2, ss_c2, ss_v2, ess0, z_ss1)
        spmm(adj_c2, adj_r2, adj_v2, z_s1, z_h2)
        spmm(ss_r2, ss_c2, ss_v2, z_ss1, z_ss2)

    @pl.when(cid == 1)
    def _():
        spmm(adj_c2, adj_r2, adj_v2, es0, z_h1)
        spmm(hh_r2, hh_c2, hh_v2, ehh0, z_hh1)
        spmm(adj_r2, adj_c2, adj_v2, z_h1, z_s2)
        spmm(hh_r2, hh_c2, hh_v2, z_hh1, z_hh2)


def _pad_edges(rows, cols, vals):
    pad = N_CHUNKS * CHUNK - E_EDGES
    r = jnp.pad(rows.astype(jnp.int32), (0, pad)).reshape(N_CHUNKS, CHUNK)
    c = jnp.pad(cols.astype(jnp.int32), (0, pad)).reshape(N_CHUNKS, CHUNK)
    v = jnp.pad(vals, (0, pad)).reshape(N_CHUNKS, CHUNK)
    return r, c, v


def _matmul_kernel(a_ref, b_ref, o_ref):
    @pl.when(pl.program_id(0) == 0)
    def _():
        o_ref[...] = jnp.zeros_like(o_ref)

    o_ref[...] += jnp.dot(a_ref[...], b_ref[...],
                          preferred_element_type=jnp.float32)


def _pallas_matmul(a, b):
    M, K = a.shape
    _, N = b.shape
    grid = (K // K_TILE,)
    return pl.pallas_call(
        _matmul_kernel,
        grid=grid,
        in_specs=[
            pl.BlockSpec((M, K_TILE), lambda k: (0, k)),
            pl.BlockSpec((K_TILE, N), lambda k: (k, 0)),
        ],
        out_specs=pl.BlockSpec((M, N), lambda k: (0, 0)),
        out_shape=jax.ShapeDtypeStruct((M, N), jnp.float32),
    )(a, b)


def kernel(sids, hids, pos, neg, ps, E_s_0, E_h_0, E_ss_0, E_hh_0,
           adj_rows, adj_cols, adj_vals, ss_rows, ss_cols, ss_vals,
           hh_rows, hh_cols, hh_vals, u_mul_s, vt, v_mul_s, ut,
           bn_gamma, bn_beta):
    adj_r2, adj_c2, adj_v2 = _pad_edges(adj_rows, adj_cols, adj_vals)
    ss_r2, ss_c2, ss_v2 = _pad_edges(ss_rows, ss_cols, ss_vals)
    hh_r2, hh_c2, hh_v2 = _pad_edges(hh_rows, hh_cols, hh_vals)

    (Z_s1, Z_h1, Z_ss1, Z_hh1, Z_s2, Z_h2, Z_ss2, Z_hh2) = _sc_all(
        adj_r2, adj_c2, adj_v2, ss_r2, ss_c2, ss_v2, hh_r2, hh_c2, hh_v2,
        E_s_0, E_h_0, E_ss_0, E_hh_0)

    G_s = E_s_0 + u_mul_s @ (vt @ (E_h_0 + Z_h1))
    G_h = E_h_0 + v_mul_s @ (ut @ (E_s_0 + Z_s1))
    E_s = E_s_0 + Z_s1 + Z_s2
    E_h = E_h_0 + Z_h1 + Z_h2
    E_ss = E_ss_0 + Z_ss1 + Z_ss2
    E_hh = E_hh_0 + Z_hh1 + Z_hh2

    ps_pad = jnp.pad(ps, ((0, 0), (0, K_PAD - N_S)))
    Es_sum_pad = jnp.pad(E_s + E_ss, ((0, K_PAD - N_S), (0, 0)))
    e_synd = _pallas_matmul(ps_pad, Es_sum_pad)

    preSum = jnp.sum(ps, axis=1, keepdims=True)
    e = e_synd / preSum
    mean = jnp.mean(e, axis=0)
    var = jnp.var(e, axis=0)
    e = (e - mean) / jnp.sqrt(var + BN_EPS) * bn_gamma + bn_beta
    e = jax.nn.relu(e)
    pre = e @ (E_h + E_hh).T
    neg_score = jnp.log(jnp.sum(jnp.exp(G_s[sids] @ E_s.T / TEMP), axis=1) + 1e-08).mean()
    neg_score = neg_score + jnp.log(jnp.sum(jnp.exp(G_h[hids] @ E_h.T / TEMP), axis=1) + 1e-08).mean()
    pos_score = jnp.clip(jnp.sum(G_s[sids] * E_s[sids], axis=1) / TEMP, -5.0, 5.0).mean() \
        + jnp.clip(jnp.sum(G_h[hids] * E_h[hids], axis=1) / TEMP, -5.0, 5.0).mean()
    loss_s = -pos_score + neg_score
    s_emb = E_s[sids]
    pos_emb = E_h[pos]
    neg_emb = E_h[neg]
    pos_scores = jnp.sum(s_emb * pos_emb, axis=-1)
    neg_scores = jnp.sum(s_emb * neg_emb, axis=-1)
    loss_r = -jnp.log(jax.nn.sigmoid(pos_scores - neg_scores)).mean()
    loss_reg = jnp.float32(0.0)
    for p in [E_s_0, E_h_0, E_ss_0, E_hh_0, bn_gamma, bn_beta]:
        loss_reg = loss_reg + jnp.square(jnp.linalg.norm(p))
    loss_reg = loss_reg * LAMBDA_2
    loss = loss_r + LAMBDA_1 * loss_s + loss_reg
    return (loss, loss_r, LAMBDA_1 * loss_s, pre)


# speedup vs baseline: 4.1105x; 1.3220x over previous
import functools

import jax
import jax.numpy as jnp
from jax import lax
from jax.experimental import pallas as pl
from jax.experimental.pallas import tpu as pltpu
from jax.experimental.pallas import tpu_sc as plsc

N_S = 10000
N_H = 10000
DIM = 128
RANK = 64
LAYER = 2
TEMP = 0.2
LAMBDA_1 = 0.2
LAMBDA_2 = 1e-07
BN_EPS = 1e-05
B = 4096
BU = 1024

E_EDGES = 320000
CHUNK = 112
CHUNKS_PER_TILE = 180
N_CHUNKS = CHUNKS_PER_TILE * 16
E_PAD = N_CHUNKS * CHUNK
ROWS_MAIN = 624
ROWS_LAST = N_S - 15 * ROWS_MAIN

K_PAD = 10240
K_TILE = 1024

_MESH = plsc.VectorSubcoreMesh(core_axis_name="c", subcore_axis_name="s")


def _fill_zeros(rbuf):
    zeros16 = jnp.zeros((16,), jnp.float32)

    def body(r, _):
        for d in range(8):
            rbuf[r, pl.ds(d * 16, 16)] = zeros16
        return 0

    lax.fori_loop(0, CHUNK, body, 0)


def _do_spmm(packed, pvals, x_hbm, z_hbm, chunk_base, z_row_base,
             acc, ibufs, vbufs, rbufs, sidxs, svals,
             isems, vsems, gsems, ssems, sid):
    base_row = sid * ROWS_MAIN
    rbuf0 = rbufs[0]

    _fill_zeros(rbuf0)

    @pl.when(sid < 15)
    def _():
        for k in range(5):
            pltpu.sync_copy(rbuf0, acc.at[pl.ds(base_row + k * CHUNK, CHUNK)])
        rem = ROWS_MAIN - 5 * CHUNK
        pltpu.sync_copy(rbuf0.at[pl.ds(0, rem)],
                        acc.at[pl.ds(base_row + 5 * CHUNK, rem)])

    @pl.when(sid == 15)
    def _():
        base = 15 * ROWS_MAIN
        for k in range(5):
            pltpu.sync_copy(rbuf0, acc.at[pl.ds(base + k * CHUNK, CHUNK)])
        rem = ROWS_LAST - 5 * CHUNK
        pltpu.sync_copy(rbuf0.at[pl.ds(0, rem)],
                        acc.at[pl.ds(base + 5 * CHUNK, rem)])

    plsc.subcore_barrier()

    base_chunk = chunk_base + sid * CHUNKS_PER_TILE
    n = CHUNKS_PER_TILE

    def idx_start(j, p2):
        pltpu.async_copy(packed.at[base_chunk + j], ibufs[p2], isems[p2])
        pltpu.async_copy(pvals.at[base_chunk + j], vbufs[p2], vsems[p2])

    def idx_wait(j, p2):
        pltpu.make_async_copy(packed.at[base_chunk + j], ibufs[p2],
                              isems[p2]).wait()
        pltpu.make_async_copy(pvals.at[base_chunk + j], vbufs[p2],
                              vsems[p2]).wait()

    def gather_start(j, p2, p3):
        pltpu.async_copy(x_hbm.at[ibufs[p2].at[0]], rbufs[p3], gsems[p3])

    def gather_wait(p2, p3):
        pltpu.make_async_copy(x_hbm.at[ibufs[p2].at[0]], rbufs[p3],
                              gsems[p3]).wait()

    def scatter_start(p3):
        pltpu.async_copy(rbufs[p3], acc.at[sidxs[p3]], ssems[p3], add=True)

    def scatter_wait(p3):
        pltpu.make_async_copy(rbufs[p3], acc.at[sidxs[p3]],
                              ssems[p3]).wait()

    def half(j, k):
        p3 = k % 3
        p2 = k % 2
        rbuf, sidx, sval = rbufs[p3], sidxs[p3], svals[p3]
        ibuf, vbuf = ibufs[p2], vbufs[p2]

        @pl.when(j + 1 < n)
        def _():
            idx_wait(j + 1, (k + 1) % 2)

        @pl.when(j >= 2)
        def _():
            scatter_wait((k + 1) % 3)

        @pl.when(j + 1 < n)
        def _():
            gather_start(j + 1, (k + 1) % 2, (k + 1) % 3)

        gather_wait(p2, p3)
        for g in range(CHUNK // 16):
            sl = pl.ds(g * 16, 16)
            sidx[sl] = ibuf[1, sl]
            sval[sl] = vbuf[sl]

        @pl.when(j + 2 < n)
        def _():
            idx_start(j + 2, k % 2)

        def scale_body(g, _):
            vv = sval[pl.ds(g * 16, 16)]
            for l in range(16):
                v = vv[l]
                e = g * 16 + l
                for d in range(8):
                    sl = pl.ds(d * 16, 16)
                    rbuf[e, sl] = rbuf[e, sl] * v
            return 0

        lax.fori_loop(0, CHUNK // 16, scale_body, 0)
        scatter_start(p3)

    idx_start(0, 0)
    idx_wait(0, 0)
    gather_start(0, 0, 0)
    idx_start(1, 1)

    def chunk_body(i6, _):
        for k in range(6):
            half(6 * i6 + k, k)
        return 0

    lax.fori_loop(0, n // 6, chunk_body, 0)
    scatter_wait((n - 2) % 3)
    scatter_wait((n - 1) % 3)
    plsc.subcore_barrier()

    zb = z_row_base + base_row

    @pl.when(sid < 15)
    def _():
        for k in range(5):
            pltpu.sync_copy(acc.at[pl.ds(base_row + k * CHUNK, CHUNK)],
                            z_hbm.at[pl.ds(zb + k * CHUNK, CHUNK)])
        rem = ROWS_MAIN - 5 * CHUNK
        pltpu.sync_copy(acc.at[pl.ds(base_row + 5 * CHUNK, rem)],
                        z_hbm.at[pl.ds(zb + 5 * CHUNK, rem)])

    @pl.when(sid == 15)
    def _():
        base = 15 * ROWS_MAIN
        zbase = z_row_base + base
        for k in range(5):
            pltpu.sync_copy(acc.at[pl.ds(base + k * CHUNK, CHUNK)],
                            z_hbm.at[pl.ds(zbase + k * CHUNK, CHUNK)])
        rem = ROWS_LAST - 5 * CHUNK
        pltpu.sync_copy(acc.at[pl.ds(base + 5 * CHUNK, rem)],
                        z_hbm.at[pl.ds(zbase + 5 * CHUNK, rem)])

    plsc.subcore_barrier()


@functools.partial(
    pl.kernel,
    mesh=_MESH,
    out_type=jax.ShapeDtypeStruct((8 * N_S, DIM), jnp.float32),
    scratch_types=[
        pltpu.VMEM_SHARED((N_S, DIM), jnp.float32),
        pltpu.VMEM((2, CHUNK), jnp.int32),
        pltpu.VMEM((2, CHUNK), jnp.int32),
        pltpu.VMEM((CHUNK,), jnp.float32),
        pltpu.VMEM((CHUNK,), jnp.float32),
        pltpu.VMEM((CHUNK, DIM), jnp.float32),
        pltpu.VMEM((CHUNK, DIM), jnp.float32),
        pltpu.VMEM((CHUNK, DIM), jnp.float32),
        pltpu.VMEM((CHUNK,), jnp.int32),
        pltpu.VMEM((CHUNK,), jnp.int32),
        pltpu.VMEM((CHUNK,), jnp.int32),
        pltpu.VMEM((CHUNK,), jnp.float32),
        pltpu.VMEM((CHUNK,), jnp.float32),
        pltpu.VMEM((CHUNK,), jnp.float32),
        pltpu.SemaphoreType.DMA,
        pltpu.SemaphoreType.DMA,
        pltpu.SemaphoreType.DMA,
        pltpu.SemaphoreType.DMA,
        pltpu.SemaphoreType.DMA,
        pltpu.SemaphoreType.DMA,
        pltpu.SemaphoreType.DMA,
        pltpu.SemaphoreType.DMA,
        pltpu.SemaphoreType.DMA,
        pltpu.SemaphoreType.DMA,
    ],
)
def _sc_all(p1idx, p1vals, p2idx, p2vals, xz, zflat,
            acc, ibuf0, ibuf1, vbuf0, vbuf1, rbuf0, rbuf1, rbuf2,
            sidx0, sidx1, sidx2, sval0, sval1, sval2,
            isem0, isem1, vsem0, vsem1,
            gsem0, gsem1, gsem2, ssem0, ssem1, ssem2):
    cid = lax.axis_index("c")
    sid = lax.axis_index("s")
    ibufs = (ibuf0, ibuf1)
    vbufs = (vbuf0, vbuf1)
    rbufs = (rbuf0, rbuf1, rbuf2)
    sidxs = (sidx0, sidx1, sidx2)
    svals = (sval0, sval1, sval2)
    isems = (isem0, isem1)
    vsems = (vsem0, vsem1)
    gsems = (gsem0, gsem1, gsem2)
    ssems = (ssem0, ssem1, ssem2)

    def spmm(packed, pvals, x_hbm, chunk_base, z_row_base):
        _do_spmm(packed, pvals, x_hbm, zflat, chunk_base, z_row_base,
                 acc, ibufs, vbufs, rbufs, sidxs, svals,
                 isems, vsems, gsems, ssems, sid)

    def l1_body(p, _):
        use = cid * 2 + p
        z_slot = 2 * p + cid
        spmm(p1idx, p1vals, xz, use * N_CHUNKS, z_slot * N_S)
        return 0

    lax.fori_loop(0, 2, l1_body, 0)

    def l2_body(p, _):
        use = cid * 2 + p
        z_slot = jnp.where(cid == 0, 5 + p, jnp.where(p == 0, 4, 7))
        spmm(p2idx, p2vals, zflat, use * N_CHUNKS, z_slot * N_S)
        return 0

    lax.fori_loop(0, 2, l2_body, 0)


def _pack_edges(rows, cols, vals, x_slot):
    pad = E_PAD - E_EDGES
    c = jnp.pad(cols.astype(jnp.int32) + x_slot * N_S,
                (0, pad)).reshape(N_CHUNKS, 1, CHUNK)
    r = jnp.pad(rows.astype(jnp.int32), (0, pad)).reshape(N_CHUNKS, 1, CHUNK)
    v = jnp.pad(vals, (0, pad)).reshape(N_CHUNKS, CHUNK)
    return jnp.concatenate([c, r], axis=1), v


def _matmul_kernel(a_ref, b_ref, o_ref):
    @pl.when(pl.program_id(0) == 0)
    def _():
        o_ref[...] = jnp.zeros_like(o_ref)

    o_ref[...] += jnp.dot(a_ref[...], b_ref[...],
                          preferred_element_type=jnp.float32)


def _pallas_matmul(a, b):
    M, K = a.shape
    _, N = b.shape
    grid = (K // K_TILE,)
    return pl.pallas_call(
        _matmul_kernel,
        grid=grid,
        in_specs=[
            pl.BlockSpec((M, K_TILE), lambda k: (0, k)),
            pl.BlockSpec((K_TILE, N), lambda k: (k, 0)),
        ],
        out_specs=pl.BlockSpec((M, N), lambda k: (0, 0)),
        out_shape=jax.ShapeDtypeStruct((M, N), jnp.float32),
    )(a, b)


def kernel(sids, hids, pos, neg, ps, E_s_0, E_h_0, E_ss_0, E_hh_0,
           adj_rows, adj_cols, adj_vals, ss_rows, ss_cols, ss_vals,
           hh_rows, hh_cols, hh_vals, u_mul_s, vt, v_mul_s, ut,
           bn_gamma, bn_beta):
    xz = jnp.concatenate([E_s_0, E_h_0, E_ss_0, E_hh_0], axis=0)
    l1_uses = [
        _pack_edges(adj_rows, adj_cols, adj_vals, 1),
        _pack_edges(ss_rows, ss_cols, ss_vals, 2),
        _pack_edges(adj_cols, adj_rows, adj_vals, 0),
        _pack_edges(hh_rows, hh_cols, hh_vals, 3),
    ]
    l2_uses = [
        _pack_edges(adj_cols, adj_rows, adj_vals, 0),
        _pack_edges(ss_rows, ss_cols, ss_vals, 2),
        _pack_edges(adj_rows, adj_cols, adj_vals, 1),
        _pack_edges(hh_rows, hh_cols, hh_vals, 3),
    ]
    p1idx = jnp.concatenate([u[0] for u in l1_uses], axis=0)
    p1vals = jnp.concatenate([u[1] for u in l1_uses], axis=0)
    p2idx = jnp.concatenate([u[0] for u in l2_uses], axis=0)
    p2vals = jnp.concatenate([u[1] for u in l2_uses], axis=0)

    zflat = _sc_all(p1idx, p1vals, p2idx, p2vals, xz)
    (Z_s1, Z_h1, Z_ss1, Z_hh1, Z_s2, Z_h2, Z_ss2, Z_hh2) = (
        zflat[k * N_S:(k + 1) * N_S] for k in range(8))

    G_s = E_s_0 + u_mul_s @ (vt @ (E_h_0 + Z_h1))
    G_h = E_h_0 + v_mul_s @ (ut @ (E_s_0 + Z_s1))
    E_s = E_s_0 + Z_s1 + Z_s2
    E_h = E_h_0 + Z_h1 + Z_h2
    E_ss = E_ss_0 + Z_ss1 + Z_ss2
    E_hh = E_hh_0 + Z_hh1 + Z_hh2

    ps_pad = jnp.pad(ps, ((0, 0), (0, K_PAD - N_S)))
    Es_sum_pad = jnp.pad(E_s + E_ss, ((0, K_PAD - N_S), (0, 0)))
    e_synd = _pallas_matmul(ps_pad, Es_sum_pad)

    preSum = jnp.sum(ps, axis=1, keepdims=True)
    e = e_synd / preSum
    mean = jnp.mean(e, axis=0)
    var = jnp.var(e, axis=0)
    e = (e - mean) / jnp.sqrt(var + BN_EPS) * bn_gamma + bn_beta
    e = jax.nn.relu(e)
    pre = e @ (E_h + E_hh).T
    neg_score = jnp.log(jnp.sum(jnp.exp(G_s[sids] @ E_s.T / TEMP), axis=1) + 1e-08).mean()
    neg_score = neg_score + jnp.log(jnp.sum(jnp.exp(G_h[hids] @ E_h.T / TEMP), axis=1) + 1e-08).mean()
    pos_score = jnp.clip(jnp.sum(G_s[sids] * E_s[sids], axis=1) / TEMP, -5.0, 5.0).mean() \
        + jnp.clip(jnp.sum(G_h[hids] * E_h[hids], axis=1) / TEMP, -5.0, 5.0).mean()
    loss_s = -pos_score + neg_score
    s_emb = E_s[sids]
    pos_emb = E_h[pos]
    neg_emb = E_h[neg]
    pos_scores = jnp.sum(s_emb * pos_emb, axis=-1)
    neg_scores = jnp.sum(s_emb * neg_emb, axis=-1)
    loss_r = -jnp.log(jax.nn.sigmoid(pos_scores - neg_scores)).mean()
    loss_reg = jnp.float32(0.0)
    for p in [E_s_0, E_h_0, E_ss_0, E_hh_0, bn_gamma, bn_beta]:
        loss_reg = loss_reg + jnp.square(jnp.linalg.norm(p))
    loss_reg = loss_reg * LAMBDA_2
    loss = loss_r + LAMBDA_1 * loss_s + loss_reg
    return (loss, loss_r, LAMBDA_1 * loss_s, pre)
